# baseline (speedup 1.0000x reference)
import functools

import jax
import jax.numpy as jnp
from jax import lax
from jax.experimental import pallas as pl
from jax.experimental.pallas import tpu as pltpu
from jax.experimental.pallas import tpu_sc as plsc

N = 10000
E = 320000
DF = 128
EMB = 64
G = 256
NC = 2
NS = 16
NW = NC * NS
NPAD = 10240
DEG_PAD = NPAD
E_PAD = 327680
EPT = E_PAD // NW
BLK = 128
NBLK = EPT // BLK
NB = 8

_mesh = plsc.VectorSubcoreMesh(core_axis_name="c", subcore_axis_name="s")


@functools.partial(
    pl.kernel,
    out_type=jax.ShapeDtypeStruct((NC * DEG_PAD,), jnp.float32),
    mesh=_mesh,
    compiler_params=pltpu.CompilerParams(use_tc_tiling_on_sc=False, needs_layout_passes=False),
    scratch_types=[
        pltpu.VMEM((NBLK, BLK), jnp.int32),
        pltpu.VMEM((BLK,), jnp.float32),
        pltpu.VMEM_SHARED((DEG_PAD,), jnp.float32),
    ] + [pltpu.SemaphoreType.DMA] * NB,
)
def _sc_deg(cols_hbm, half_hbm, deg_out, cbig, obuf, deg_sh, *dsem):
    c = lax.axis_index("c")
    s = lax.axis_index("s")
    wid = s * NC + c
    for k in range(BLK // 16):
        obuf[pl.ds(16 * k, 16)] = jnp.ones((16,), jnp.float32)
    pltpu.sync_copy(cols_hbm.at[pl.ds(wid * NBLK, NBLK)], cbig)
    pltpu.sync_copy(half_hbm.at[pl.ds(s * 640, 640)], deg_sh.at[pl.ds(s * 640, 640)])
    plsc.subcore_barrier()

    for b in range(NB):
        pltpu.async_copy(obuf, deg_sh.at[cbig.at[b]], dsem[b], add=True)

    def outer(jo, carry):
        for b in range(NB):
            j = jo * NB + b
            pltpu.make_async_copy(obuf, deg_sh.at[cbig.at[j]], dsem[b]).wait()

            @pl.when(j + NB < NBLK)
            def _():
                pltpu.async_copy(obuf, deg_sh.at[cbig.at[j + NB]], dsem[b], add=True)

        return carry

    lax.fori_loop(0, NBLK // NB, outer, 0)
    plsc.subcore_barrier()
    pltpu.sync_copy(
        deg_sh.at[pl.ds(s * 640, 640)],
        deg_out.at[pl.ds(c * DEG_PAD + s * 640, 640)],
    )


@functools.partial(
    pl.kernel,
    out_type=jax.ShapeDtypeStruct((NC, N, EMB), jnp.float32),
    mesh=_mesh,
    compiler_params=pltpu.CompilerParams(use_tc_tiling_on_sc=False, needs_layout_passes=False),
    scratch_types=[
        pltpu.VMEM((NBLK, BLK), jnp.int32),
        pltpu.VMEM((NBLK, BLK), jnp.int32),
        pltpu.VMEM((NB, BLK, EMB), jnp.float32),
        pltpu.VMEM_SHARED((NPAD, EMB), jnp.float32),
    ] + [pltpu.SemaphoreType.DMA] * (2 * NB),
)
def _sc_prop(rows_hbm, cols_hbm, p_hbm, z_hbm, t_out, rbig, cbig, gbuf, t_sh, *sems):
    gsem = sems[:NB]
    ssem = sems[NB:]
    c = lax.axis_index("c")
    s = lax.axis_index("s")
    wid = s * NC + c
    pltpu.sync_copy(rows_hbm.at[pl.ds(wid * NBLK, NBLK)], rbig)
    pltpu.sync_copy(cols_hbm.at[pl.ds(wid * NBLK, NBLK)], cbig)
    nq_slc = jnp.where(s == NS - 1, 5, 8)

    def initc(q, carry):
        r0 = s * 640 + q * 80
        pltpu.sync_copy(z_hbm, t_sh.at[pl.ds(r0, 80)])
        return carry

    lax.fori_loop(0, nq_slc, initc, 0)
    plsc.subcore_barrier()

    for b in range(NB):
        pltpu.async_copy(p_hbm.at[rbig.at[b]], gbuf.at[b], gsem[b])

    def outer(jo, carry):
        for b in range(NB):
            j = jo * NB + b
            pltpu.make_async_copy(p_hbm.at[rbig.at[j]], gbuf.at[b], gsem[b]).wait()
            pltpu.async_copy(gbuf.at[b], t_sh.at[cbig.at[j]], ssem[b], add=True)

            @pl.when(j + NB < NBLK)
            def _():
                pltpu.make_async_copy(gbuf.at[b], t_sh.at[cbig.at[j]], ssem[b]).wait()
                pltpu.async_copy(p_hbm.at[rbig.at[j + NB]], gbuf.at[b], gsem[b])

        return carry

    lax.fori_loop(0, NBLK // NB, outer, 0)
    for b in range(NB):
        pltpu.make_async_copy(gbuf.at[b], t_sh.at[cbig.at[NBLK - NB + b]], ssem[b]).wait()
    plsc.subcore_barrier()

    def outc(q, carry):
        r0 = s * 640 + q * 80
        pltpu.sync_copy(t_sh.at[pl.ds(r0, 80)], t_out.at[c, pl.ds(r0, 80)])
        return carry

    lax.fori_loop(0, nq_slc, outc, 0)


POOL_CH = 320
POOL_Q = 80


@functools.partial(
    pl.kernel,
    out_type=[
        jax.ShapeDtypeStruct((NW, G, EMB), jnp.float32),
        jax.ShapeDtypeStruct((NW, G, EMB), jnp.float32),
        jax.ShapeDtypeStruct((NW, G), jnp.float32),
    ],
    mesh=_mesh,
    compiler_params=pltpu.CompilerParams(use_tc_tiling_on_sc=False, needs_layout_passes=False),
    scratch_types=[
        pltpu.VMEM((POOL_CH, EMB), jnp.float32),
        pltpu.VMEM((POOL_CH,), jnp.int32),
        pltpu.VMEM((G, EMB), jnp.float32),
        pltpu.VMEM((G, EMB), jnp.float32),
        pltpu.VMEM((G,), jnp.float32),
    ],
)
def _sc_pool(h_hbm, bi_hbm, maxp, sump, cntp, hbuf, bibuf, maxb, sumb, cntb):
    c = lax.axis_index("c")
    s = lax.axis_index("s")
    wid = s * NC + c
    base = wid * POOL_CH
    last = wid == NW - 1
    nq = jnp.where(last, 1, POOL_CH // POOL_Q)
    ngrp = jnp.where(last, POOL_Q // 16, POOL_CH // 16)

    def loadq(q, carry):
        off = q * POOL_Q
        pltpu.sync_copy(h_hbm.at[pl.ds(base + off, POOL_Q)], hbuf.at[pl.ds(off, POOL_Q)])
        pltpu.sync_copy(bi_hbm.at[pl.ds(base + off, POOL_Q)], bibuf.at[pl.ds(off, POOL_Q)])
        return carry

    lax.fori_loop(0, nq, loadq, 0)

    neg = jnp.full((16,), -jnp.inf, jnp.float32)
    zero = jnp.zeros((16,), jnp.float32)

    def initb(t, carry):
        r = t // (EMB // 16)
        f = t % (EMB // 16)
        maxb[r, pl.ds(16 * f, 16)] = neg
        sumb[r, pl.ds(16 * f, 16)] = zero
        return carry

    lax.fori_loop(0, G * EMB // 16, initb, 0)
    for k in range(G // 16):
        cntb[pl.ds(16 * k, 16)] = zero

    lanes = jnp.arange(16, dtype=jnp.int32)
    lane0 = lanes == 0
    one16 = jnp.ones((16,), jnp.float32)

    def grp(jj, carry):
        gvec = bibuf[pl.ds(16 * jj, 16)]
        for k in range(16):
            j = 16 * jj + k
            g = gvec[k]
            gv = jnp.full((16,), g, jnp.int32)
            plsc.addupdate_scatter(cntb, [gv], one16, mask=lane0)
            for f in range(EMB // 16):
                fv = 16 * f + lanes
                hv = hbuf[j, pl.ds(16 * f, 16)]
                mv = plsc.load_gather(maxb, [gv, fv])
                plsc.store_scatter(maxb, [gv, fv], jnp.maximum(mv, hv))
                plsc.addupdate_scatter(sumb, [gv, fv], hv)
        return carry

    lax.fori_loop(0, ngrp, grp, 0)
    pltpu.sync_copy(maxb, maxp.at[wid])
    pltpu.sync_copy(sumb, sump.at[wid])
    pltpu.sync_copy(cntb, cntp.at[wid])


def _tc_pre_body(deg_ref, x_ref, w_ref, p_ref, dinv_ref):
    deg = deg_ref[0, :N] + deg_ref[1, :N]
    dinv = lax.rsqrt(deg)
    p = jnp.dot(x_ref[...], w_ref[...], preferred_element_type=jnp.float32)
    p_ref[...] = p * dinv[:, None]
    dinv_ref[...] = dinv


def _tc_pre(degp, x, w):
    return pl.pallas_call(
        _tc_pre_body,
        out_shape=[
            jax.ShapeDtypeStruct((N, EMB), jnp.float32),
            jax.ShapeDtypeStruct((N,), jnp.float32),
        ],
    )(degp, x, w)


def _tc_mid_body(t_ref, pp_ref, dinv_ref, b_ref, w_ref, p_ref):
    t = t_ref[0] + t_ref[1] + pp_ref[...]
    dinv = dinv_ref[...][:, None]
    h = jnp.tanh(t * dinv + b_ref[...][None, :])
    p = jnp.dot(h, w_ref[...], preferred_element_type=jnp.float32)
    p_ref[...] = p * dinv


def _tc_mid(tp, p_prev, dinv, b, w):
    return pl.pallas_call(
        _tc_mid_body,
        out_shape=jax.ShapeDtypeStruct((N, EMB), jnp.float32),
    )(tp, p_prev, dinv, b, w)


def _tc_fin_body(t_ref, pp_ref, dinv_ref, b_ref, h_ref):
    t = t_ref[0] + t_ref[1] + pp_ref[...]
    h_ref[...] = jnp.tanh(t * dinv_ref[...][:, None] + b_ref[...][None, :])


def _tc_fin(tp, p_prev, dinv, b):
    return pl.pallas_call(
        _tc_fin_body,
        out_shape=jax.ShapeDtypeStruct((N, EMB), jnp.float32),
    )(tp, p_prev, dinv, b)


def _tc_post_body(maxp_ref, sump_ref, cntp_ref, wout_ref, bout_ref, out_ref, hid_ref):
    gmax = jnp.max(maxp_ref[...], axis=0)
    gsum = jnp.sum(sump_ref[...], axis=0)
    cnt = jnp.sum(cntp_ref[...], axis=0)
    gmean = gsum / jnp.clip(cnt, 1.0)[:, None]
    hidden = jnp.concatenate([gmax, gmean], axis=1)
    hid_ref[...] = hidden
    out_ref[...] = (
        jnp.dot(hidden, wout_ref[...], preferred_element_type=jnp.float32)
        + bout_ref[...][None, :]
    )


def _tc_post(maxp, sump, cntp, wout, bout):
    return pl.pallas_call(
        _tc_post_body,
        out_shape=[
            jax.ShapeDtypeStruct((G, 1), jnp.float32),
            jax.ShapeDtypeStruct((G, 2 * EMB), jnp.float32),
        ],
    )(maxp, sump, cntp, wout, bout)


def _tc_edges_body(e_ref, r_ref, c_ref):
    rows = e_ref[0, :].reshape(E // BLK, BLK)
    cols = e_ref[1, :].reshape(E // BLK, BLK)
    npad = E_PAD - E
    padr = (jnp.arange(npad, dtype=jnp.int32) % N).reshape(npad // BLK, BLK)
    padc = N + (jnp.arange(npad, dtype=jnp.int32) % (NPAD - N)).reshape(npad // BLK, BLK)
    r_ref[...] = jnp.concatenate([rows, padr], axis=0)
    c_ref[...] = jnp.concatenate([cols, padc], axis=0)


def _tc_edges(edge_index):
    return pl.pallas_call(
        _tc_edges_body,
        out_shape=[
            jax.ShapeDtypeStruct((E_PAD // BLK, BLK), jnp.int32),
            jax.ShapeDtypeStruct((E_PAD // BLK, BLK), jnp.int32),
        ],
    )(edge_index)


@jax.jit
def kernel(x, edge_index, batch_index, W0, b0, W1, b1, W2, b2, Wout, bout):
    rows, cols = _tc_edges(edge_index)
    half = jnp.full((DEG_PAD,), 0.5, jnp.float32)
    zc = jnp.zeros((80, EMB), jnp.float32)
    degp = _sc_deg(cols, half).reshape(NC, DEG_PAD)
    P0, dinv = _tc_pre(degp, x, W0)
    T0 = _sc_prop(rows, cols, P0, zc)
    P1 = _tc_mid(T0, P0, dinv, b0, W1)
    T1 = _sc_prop(rows, cols, P1, zc)
    P2 = _tc_mid(T1, P1, dinv, b1, W2)
    T2 = _sc_prop(rows, cols, P2, zc)
    h3 = _tc_fin(T2, P2, dinv, b2)
    maxp, sump, cntp = _sc_pool(h3, batch_index)
    out, hidden = _tc_post(maxp, sump, cntp, Wout, bout)
    return out, hidden


# speedup vs baseline: 1.1473x; 1.1473x over previous
import functools

import jax
import jax.numpy as jnp
from jax import lax
from jax.experimental import pallas as pl
from jax.experimental.pallas import tpu as pltpu
from jax.experimental.pallas import tpu_sc as plsc

N = 10000
E = 320000
DF = 128
EMB = 64
G = 256
NC = 2
NS = 16
NW = NC * NS
NPAD = 10240
DEG_PAD = 10240
E_PAD = 327680
EPT = E_PAD // NW
BLK = 128
NBLK = EPT // BLK
NB = 8

_mesh = plsc.VectorSubcoreMesh(core_axis_name="c", subcore_axis_name="s")


@functools.partial(
    pl.kernel,
    out_type=jax.ShapeDtypeStruct((NC * DEG_PAD,), jnp.float32),
    mesh=_mesh,
    compiler_params=pltpu.CompilerParams(use_tc_tiling_on_sc=False, needs_layout_passes=False),
    scratch_types=[
        pltpu.VMEM((NBLK, BLK), jnp.int32),
        pltpu.VMEM((BLK,), jnp.float32),
        pltpu.VMEM_SHARED((DEG_PAD,), jnp.float32),
    ] + [pltpu.SemaphoreType.DMA] * NB,
)
def _sc_deg(cols_hbm, half_hbm, deg_out, cbig, obuf, deg_sh, *dsem):
    c = lax.axis_index("c")
    s = lax.axis_index("s")
    wid = s * NC + c
    for k in range(BLK // 16):
        obuf[pl.ds(16 * k, 16)] = jnp.ones((16,), jnp.float32)
    pltpu.sync_copy(cols_hbm.at[pl.ds(wid * NBLK, NBLK)], cbig)
    pltpu.sync_copy(half_hbm.at[pl.ds(s * 640, 640)], deg_sh.at[pl.ds(s * 640, 640)])
    plsc.subcore_barrier()

    for b in range(NB):
        pltpu.async_copy(obuf, deg_sh.at[cbig.at[b]], dsem[b], add=True)

    def outer(jo, carry):
        for b in range(NB):
            j = jo * NB + b
            pltpu.make_async_copy(obuf, deg_sh.at[cbig.at[j]], dsem[b]).wait()

            @pl.when(j + NB < NBLK)
            def _():
                pltpu.async_copy(obuf, deg_sh.at[cbig.at[j + NB]], dsem[b], add=True)

        return carry

    lax.fori_loop(0, NBLK // NB, outer, 0)
    plsc.subcore_barrier()
    pltpu.sync_copy(
        deg_sh.at[pl.ds(s * 640, 640)],
        deg_out.at[pl.ds(c * DEG_PAD + s * 640, 640)],
    )


@functools.partial(
    pl.kernel,
    out_type=jax.ShapeDtypeStruct((NC, N, EMB), jnp.float32),
    mesh=_mesh,
    compiler_params=pltpu.CompilerParams(use_tc_tiling_on_sc=False, needs_layout_passes=False),
    scratch_types=[
        pltpu.VMEM((NBLK, BLK), jnp.int32),
        pltpu.VMEM((NBLK, BLK), jnp.int32),
        pltpu.VMEM((NB, BLK, EMB), jnp.float32),
        pltpu.VMEM_SHARED((NPAD, EMB), jnp.float32),
    ] + [pltpu.SemaphoreType.DMA] * (2 * NB),
)
def _sc_prop(rows_hbm, cols_hbm, p_hbm, t_out, rbig, cbig, gbuf, t_sh, *sems):
    gsem = sems[:NB]
    ssem = sems[NB:]
    c = lax.axis_index("c")
    s = lax.axis_index("s")
    wid = s * NC + c
    pltpu.sync_copy(rows_hbm.at[pl.ds(wid * NBLK, NBLK)], rbig)
    pltpu.sync_copy(cols_hbm.at[pl.ds(wid * NBLK, NBLK)], cbig)
    nq_slc = jnp.where(s == NS - 1, 5, 8)

    zero16 = jnp.zeros((16,), jnp.float32)

    def zrow(r, carry):
        for f in range(EMB // 16):
            gbuf[0, r, pl.ds(16 * f, 16)] = zero16
        return carry

    lax.fori_loop(0, 80, zrow, 0)
    zsrc = gbuf.at[0].at[pl.ds(0, 80)]

    def initc(q, carry):
        r0 = s * 640 + q * 80
        pltpu.sync_copy(zsrc, t_sh.at[pl.ds(r0, 80)])
        return carry

    lax.fori_loop(0, nq_slc, initc, 0)
    plsc.subcore_barrier()

    for b in range(NB):
        pltpu.async_copy(p_hbm.at[rbig.at[b]], gbuf.at[b], gsem[b])

    def outer(jo, carry):
        for b in range(NB):
            j = jo * NB + b
            pltpu.make_async_copy(p_hbm.at[rbig.at[j]], gbuf.at[b], gsem[b]).wait()
            pltpu.async_copy(gbuf.at[b], t_sh.at[cbig.at[j]], ssem[b], add=True)

            @pl.when(j + NB < NBLK)
            def _():
                pltpu.make_async_copy(gbuf.at[b], t_sh.at[cbig.at[j]], ssem[b]).wait()
                pltpu.async_copy(p_hbm.at[rbig.at[j + NB]], gbuf.at[b], gsem[b])

        return carry

    lax.fori_loop(0, NBLK // NB, outer, 0)
    for b in range(NB):
        pltpu.make_async_copy(gbuf.at[b], t_sh.at[cbig.at[NBLK - NB + b]], ssem[b]).wait()
    plsc.subcore_barrier()

    def outc(q, carry):
        r0 = s * 640 + q * 80
        pltpu.sync_copy(t_sh.at[pl.ds(r0, 80)], t_out.at[c, pl.ds(r0, 80)])
        return carry

    lax.fori_loop(0, nq_slc, outc, 0)


POOL_CH = 320
POOL_Q = 80


@functools.partial(
    pl.kernel,
    out_type=[
        jax.ShapeDtypeStruct((NW, G, EMB), jnp.float32),
        jax.ShapeDtypeStruct((NW, G, EMB), jnp.float32),
        jax.ShapeDtypeStruct((NW, G), jnp.float32),
    ],
    mesh=_mesh,
    compiler_params=pltpu.CompilerParams(use_tc_tiling_on_sc=False, needs_layout_passes=False),
    scratch_types=[
        pltpu.VMEM((POOL_CH, EMB), jnp.float32),
        pltpu.VMEM((POOL_CH,), jnp.int32),
        pltpu.VMEM((G, EMB), jnp.float32),
        pltpu.VMEM((G, EMB), jnp.float32),
        pltpu.VMEM((G,), jnp.float32),
    ],
)
def _sc_pool(h_hbm, bi_hbm, maxp, sump, cntp, hbuf, bibuf, maxb, sumb, cntb):
    c = lax.axis_index("c")
    s = lax.axis_index("s")
    wid = s * NC + c
    base = wid * POOL_CH
    last = wid == NW - 1
    nq = jnp.where(last, 1, POOL_CH // POOL_Q)
    ngrp = jnp.where(last, POOL_Q // 16, POOL_CH // 16)

    def loadq(q, carry):
        off = q * POOL_Q
        pltpu.sync_copy(h_hbm.at[pl.ds(base + off, POOL_Q)], hbuf.at[pl.ds(off, POOL_Q)])
        pltpu.sync_copy(bi_hbm.at[pl.ds(base + off, POOL_Q)], bibuf.at[pl.ds(off, POOL_Q)])
        return carry

    lax.fori_loop(0, nq, loadq, 0)

    neg = jnp.full((16,), -jnp.inf, jnp.float32)
    zero = jnp.zeros((16,), jnp.float32)

    def initb(t, carry):
        r = t // (EMB // 16)
        f = t % (EMB // 16)
        maxb[r, pl.ds(16 * f, 16)] = neg
        sumb[r, pl.ds(16 * f, 16)] = zero
        return carry

    lax.fori_loop(0, G * EMB // 16, initb, 0)
    for k in range(G // 16):
        cntb[pl.ds(16 * k, 16)] = zero

    lanes = jnp.arange(16, dtype=jnp.int32)
    lane0 = lanes == 0
    one16 = jnp.ones((16,), jnp.float32)

    def grp(jj, carry):
        gvec = bibuf[pl.ds(16 * jj, 16)]
        for k in range(16):
            j = 16 * jj + k
            g = gvec[k]
            gv = jnp.full((16,), g, jnp.int32)
            plsc.addupdate_scatter(cntb, [gv], one16, mask=lane0)
            for f in range(EMB // 16):
                fv = 16 * f + lanes
                hv = hbuf[j, pl.ds(16 * f, 16)]
                mv = plsc.load_gather(maxb, [gv, fv])
                plsc.store_scatter(maxb, [gv, fv], jnp.maximum(mv, hv))
                plsc.addupdate_scatter(sumb, [gv, fv], hv)
        return carry

    lax.fori_loop(0, ngrp, grp, 0)
    pltpu.sync_copy(maxb, maxp.at[wid])
    pltpu.sync_copy(sumb, sump.at[wid])
    pltpu.sync_copy(cntb, cntp.at[wid])


def _tc_pre_body(deg_ref, x_ref, w_ref, p_ref, dinv_ref):
    deg = deg_ref[0, :N] + deg_ref[1, :N]
    dinv = lax.rsqrt(deg)
    p = jnp.dot(x_ref[...], w_ref[...], preferred_element_type=jnp.float32)
    p_ref[...] = p * dinv[:, None]
    dinv_ref[...] = dinv


def _tc_pre(degp, x, w):
    return pl.pallas_call(
        _tc_pre_body,
        out_shape=[
            jax.ShapeDtypeStruct((N, EMB), jnp.float32),
            jax.ShapeDtypeStruct((N,), jnp.float32),
        ],
    )(degp, x, w)


def _tc_mid_body(t_ref, pp_ref, dinv_ref, b_ref, w_ref, p_ref):
    t = t_ref[0] + t_ref[1] + pp_ref[...]
    dinv = dinv_ref[...][:, None]
    h = jnp.tanh(t * dinv + b_ref[...][None, :])
    p = jnp.dot(h, w_ref[...], preferred_element_type=jnp.float32)
    p_ref[...] = p * dinv


def _tc_mid(tp, p_prev, dinv, b, w):
    return pl.pallas_call(
        _tc_mid_body,
        out_shape=jax.ShapeDtypeStruct((N, EMB), jnp.float32),
    )(tp, p_prev, dinv, b, w)


def _tc_fin_body(t_ref, pp_ref, dinv_ref, b_ref, h_ref):
    t = t_ref[0] + t_ref[1] + pp_ref[...]
    h_ref[...] = jnp.tanh(t * dinv_ref[...][:, None] + b_ref[...][None, :])


def _tc_fin(tp, p_prev, dinv, b):
    return pl.pallas_call(
        _tc_fin_body,
        out_shape=jax.ShapeDtypeStruct((N, EMB), jnp.float32),
    )(tp, p_prev, dinv, b)


def _tc_post_body(maxp_ref, sump_ref, cntp_ref, wout_ref, bout_ref, out_ref, hid_ref):
    gmax = jnp.max(maxp_ref[...], axis=0)
    gsum = jnp.sum(sump_ref[...], axis=0)
    cnt = jnp.sum(cntp_ref[...], axis=0)
    gmean = gsum / jnp.clip(cnt, 1.0)[:, None]
    hidden = jnp.concatenate([gmax, gmean], axis=1)
    hid_ref[...] = hidden
    out_ref[...] = (
        jnp.dot(hidden, wout_ref[...], preferred_element_type=jnp.float32)
        + bout_ref[...][None, :]
    )


def _tc_post(maxp, sump, cntp, wout, bout):
    return pl.pallas_call(
        _tc_post_body,
        out_shape=[
            jax.ShapeDtypeStruct((G, 1), jnp.float32),
            jax.ShapeDtypeStruct((G, 2 * EMB), jnp.float32),
        ],
    )(maxp, sump, cntp, wout, bout)


def _tc_edges_body(e_ref, r_ref, c_ref):
    rows = e_ref[0, :].reshape(E // BLK, BLK)
    cols = e_ref[1, :].reshape(E // BLK, BLK)
    npad = E_PAD - E
    padr = (jnp.arange(npad, dtype=jnp.int32) % N).reshape(npad // BLK, BLK)
    padc = N + (jnp.arange(npad, dtype=jnp.int32) % (NPAD - N)).reshape(npad // BLK, BLK)
    r_ref[...] = jnp.concatenate([rows, padr], axis=0)
    c_ref[...] = jnp.concatenate([cols, padc], axis=0)


def _tc_edges(edge_index):
    return pl.pallas_call(
        _tc_edges_body,
        out_shape=[
            jax.ShapeDtypeStruct((E_PAD // BLK, BLK), jnp.int32),
            jax.ShapeDtypeStruct((E_PAD // BLK, BLK), jnp.int32),
        ],
    )(edge_index)


@jax.jit
def kernel(x, edge_index, batch_index, W0, b0, W1, b1, W2, b2, Wout, bout):
    rows, cols = _tc_edges(edge_index)
    half = jnp.full((DEG_PAD,), 0.5, jnp.float32)
    degp = _sc_deg(cols, half).reshape(NC, DEG_PAD)
    P0, dinv = _tc_pre(degp, x, W0)
    T0 = _sc_prop(rows, cols, P0)
    P1 = _tc_mid(T0, P0, dinv, b0, W1)
    T1 = _sc_prop(rows, cols, P1)
    P2 = _tc_mid(T1, P1, dinv, b1, W2)
    T2 = _sc_prop(rows, cols, P2)
    h3 = _tc_fin(T2, P2, dinv, b2)
    maxp, sump, cntp = _sc_pool(h3, batch_index)
    out, hidden = _tc_post(maxp, sump, cntp, Wout, bout)
    return out, hidden


# speedup vs baseline: 1.3677x; 1.1921x over previous
import functools

import jax
import jax.numpy as jnp
from jax import lax
from jax.experimental import pallas as pl
from jax.experimental.pallas import tpu as pltpu
from jax.experimental.pallas import tpu_sc as plsc

N = 10000
E = 320000
DF = 128
EMB = 64
G = 256
NC = 2
NS = 16
NW = NC * NS
NPAD = 10240
DEG_PAD = 10240
E_PAD = 327680
EPT = E_PAD // NW
BLK = 128
NBLK = EPT // BLK
NB = 8

_mesh = plsc.VectorSubcoreMesh(core_axis_name="c", subcore_axis_name="s")


@functools.partial(
    pl.kernel,
    out_type=jax.ShapeDtypeStruct((NC * DEG_PAD,), jnp.float32),
    mesh=_mesh,
    compiler_params=pltpu.CompilerParams(use_tc_tiling_on_sc=False, needs_layout_passes=False),
    scratch_types=[
        pltpu.VMEM((NBLK, BLK), jnp.int32),
        pltpu.VMEM((BLK,), jnp.float32),
        pltpu.VMEM_SHARED((DEG_PAD,), jnp.float32),
    ] + [pltpu.SemaphoreType.DMA] * NB,
)
def _sc_deg(cols_hbm, half_hbm, deg_out, cbig, obuf, deg_sh, *dsem):
    c = lax.axis_index("c")
    s = lax.axis_index("s")
    wid = s * NC + c
    for k in range(BLK // 16):
        obuf[pl.ds(16 * k, 16)] = jnp.ones((16,), jnp.float32)
    pltpu.sync_copy(cols_hbm.at[pl.ds(wid * NBLK, NBLK)], cbig)
    pltpu.sync_copy(half_hbm.at[pl.ds(s * 640, 640)], deg_sh.at[pl.ds(s * 640, 640)])
    plsc.subcore_barrier()

    for b in range(NB):
        pltpu.async_copy(obuf, deg_sh.at[cbig.at[b]], dsem[b], add=True)

    def outer(jo, carry):
        for b in range(NB):
            j = jo * NB + b
            pltpu.make_async_copy(obuf, deg_sh.at[cbig.at[j]], dsem[b]).wait()

            @pl.when(j + NB < NBLK)
            def _():
                pltpu.async_copy(obuf, deg_sh.at[cbig.at[j + NB]], dsem[b], add=True)

        return carry

    lax.fori_loop(0, NBLK // NB, outer, 0)
    plsc.subcore_barrier()
    pltpu.sync_copy(
        deg_sh.at[pl.ds(s * 640, 640)],
        deg_out.at[pl.ds(c * DEG_PAD + s * 640, 640)],
    )


@functools.partial(
    pl.kernel,
    out_type=jax.ShapeDtypeStruct((NC, N, EMB), jnp.float32),
    mesh=_mesh,
    compiler_params=pltpu.CompilerParams(use_tc_tiling_on_sc=False, needs_layout_passes=False),
    scratch_types=[
        pltpu.VMEM((NBLK, BLK), jnp.int32),
        pltpu.VMEM((NBLK, BLK), jnp.int32),
        pltpu.VMEM((NB, BLK, EMB), jnp.float32),
        pltpu.VMEM_SHARED((NPAD, EMB), jnp.float32),
    ] + [pltpu.SemaphoreType.DMA] * (2 * NB),
)
def _sc_prop(rows_hbm, cols_hbm, p_hbm, t_out, rbig, cbig, gbuf, t_sh, *sems):
    gsem = sems[:NB]
    ssem = sems[NB:]
    c = lax.axis_index("c")
    s = lax.axis_index("s")
    wid = s * NC + c
    pltpu.sync_copy(rows_hbm.at[pl.ds(wid * NBLK, NBLK)], rbig)
    pltpu.sync_copy(cols_hbm.at[pl.ds(wid * NBLK, NBLK)], cbig)
    nq_slc = jnp.where(s == NS - 1, 5, 8)

    zero16 = jnp.zeros((16,), jnp.float32)

    def zrow(r, carry):
        for f in range(EMB // 16):
            gbuf[0, r, pl.ds(16 * f, 16)] = zero16
        return carry

    lax.fori_loop(0, 80, zrow, 0)
    zsrc = gbuf.at[0].at[pl.ds(0, 80)]

    def initc(q, carry):
        r0 = s * 640 + q * 80
        pltpu.sync_copy(zsrc, t_sh.at[pl.ds(r0, 80)])
        return carry

    lax.fori_loop(0, nq_slc, initc, 0)
    plsc.subcore_barrier()

    for b in range(NB):
        pltpu.async_copy(p_hbm.at[rbig.at[b]], gbuf.at[b], gsem[b])

    def outer(jo, carry):
        for b in range(NB):
            j = jo * NB + b
            pltpu.make_async_copy(p_hbm.at[rbig.at[j]], gbuf.at[b], gsem[b]).wait()
            pltpu.async_copy(gbuf.at[b], t_sh.at[cbig.at[j]], ssem[b], add=True)

            @pl.when(j + NB < NBLK)
            def _():
                pltpu.make_async_copy(gbuf.at[b], t_sh.at[cbig.at[j]], ssem[b]).wait()
                pltpu.async_copy(p_hbm.at[rbig.at[j + NB]], gbuf.at[b], gsem[b])

        return carry

    lax.fori_loop(0, NBLK // NB, outer, 0)
    for b in range(NB):
        pltpu.make_async_copy(gbuf.at[b], t_sh.at[cbig.at[NBLK - NB + b]], ssem[b]).wait()
    plsc.subcore_barrier()

    def outc(q, carry):
        r0 = s * 640 + q * 80
        pltpu.sync_copy(t_sh.at[pl.ds(r0, 80)], t_out.at[c, pl.ds(r0, 80)])
        return carry

    lax.fori_loop(0, nq_slc, outc, 0)


POOL_CH = 320
POOL_Q = 80


@functools.partial(
    pl.kernel,
    out_type=[
        jax.ShapeDtypeStruct((NW, G, EMB), jnp.float32),
        jax.ShapeDtypeStruct((NW, G, EMB), jnp.float32),
        jax.ShapeDtypeStruct((NW, G), jnp.float32),
    ],
    mesh=_mesh,
    compiler_params=pltpu.CompilerParams(use_tc_tiling_on_sc=False, needs_layout_passes=False),
    scratch_types=[
        pltpu.VMEM((POOL_CH, EMB), jnp.float32),
        pltpu.VMEM((POOL_CH,), jnp.int32),
        pltpu.VMEM((G, EMB), jnp.float32),
        pltpu.VMEM((G, EMB), jnp.float32),
        pltpu.VMEM((G,), jnp.float32),
    ],
)
def _sc_pool(h_hbm, bi_hbm, maxp, sump, cntp, hbuf, bibuf, maxb, sumb, cntb):
    c = lax.axis_index("c")
    s = lax.axis_index("s")
    wid = s * NC + c
    base = wid * POOL_CH
    last = wid == NW - 1
    nq = jnp.where(last, 1, POOL_CH // POOL_Q)
    ngrp = jnp.where(last, POOL_Q // 16, POOL_CH // 16)

    def loadq(q, carry):
        off = q * POOL_Q
        pltpu.sync_copy(h_hbm.at[pl.ds(base + off, POOL_Q)], hbuf.at[pl.ds(off, POOL_Q)])
        pltpu.sync_copy(bi_hbm.at[pl.ds(base + off, POOL_Q)], bibuf.at[pl.ds(off, POOL_Q)])
        return carry

    lax.fori_loop(0, nq, loadq, 0)

    neg = jnp.full((16,), -jnp.inf, jnp.float32)
    zero = jnp.zeros((16,), jnp.float32)

    def initb(t, carry):
        r = t // (EMB // 16)
        f = t % (EMB // 16)
        maxb[r, pl.ds(16 * f, 16)] = neg
        sumb[r, pl.ds(16 * f, 16)] = zero
        return carry

    lax.fori_loop(0, G * EMB // 16, initb, 0)
    for k in range(G // 16):
        cntb[pl.ds(16 * k, 16)] = zero

    lanes = jnp.arange(16, dtype=jnp.int32)
    lane0 = lanes == 0
    one16 = jnp.ones((16,), jnp.float32)

    def grp(jj, carry):
        gvec = bibuf[pl.ds(16 * jj, 16)]
        for k in range(16):
            j = 16 * jj + k
            g = gvec[k]
            gv = jnp.full((16,), g, jnp.int32)
            plsc.addupdate_scatter(cntb, [gv], one16, mask=lane0)
            for f in range(EMB // 16):
                fv = 16 * f + lanes
                hv = hbuf[j, pl.ds(16 * f, 16)]
                mv = plsc.load_gather(maxb, [gv, fv])
                plsc.store_scatter(maxb, [gv, fv], jnp.maximum(mv, hv))
                plsc.addupdate_scatter(sumb, [gv, fv], hv)
        return carry

    lax.fori_loop(0, ngrp, grp, 0)
    pltpu.sync_copy(maxb, maxp.at[wid])
    pltpu.sync_copy(sumb, sump.at[wid])
    pltpu.sync_copy(cntb, cntp.at[wid])


def _blockdiag(w):
    z = jnp.zeros_like(w[:, :EMB])
    top = jnp.concatenate([w, z], axis=1)
    bot = jnp.concatenate([z, w], axis=1)
    return jnp.concatenate([top, bot], axis=0)


def _tc_pre_body(deg_ref, xp_ref, w_ref, p_ref, dinvp_ref):
    deg2 = deg_ref[0, : N // 2, :] + deg_ref[1, : N // 2, :]
    di2 = lax.rsqrt(deg2)
    dinvp = jnp.concatenate(
        [jnp.broadcast_to(di2[:, 0:1], (N // 2, EMB)),
         jnp.broadcast_to(di2[:, 1:2], (N // 2, EMB))], axis=1)
    wbd = _blockdiag(w_ref[...])
    p = jnp.dot(xp_ref[...], wbd, preferred_element_type=jnp.float32)
    p_ref[...] = p * dinvp
    dinvp_ref[...] = dinvp


def _tc_pre(degp2, xp, w):
    return pl.pallas_call(
        _tc_pre_body,
        out_shape=[
            jax.ShapeDtypeStruct((N // 2, 2 * EMB), jnp.float32),
            jax.ShapeDtypeStruct((N // 2, 2 * EMB), jnp.float32),
        ],
    )(degp2, xp, w)


def _tc_mid_body(t_ref, pp_ref, dinvp_ref, b_ref, w_ref, p_ref):
    t = t_ref[0] + t_ref[1] + pp_ref[...]
    dinvp = dinvp_ref[...]
    bp = jnp.concatenate([b_ref[...], b_ref[...]])
    h = jnp.tanh(t * dinvp + bp[None, :])
    wbd = _blockdiag(w_ref[...])
    p = jnp.dot(h, wbd, preferred_element_type=jnp.float32)
    p_ref[...] = p * dinvp


def _tc_mid(tp, p_prev, dinv, b, w):
    return pl.pallas_call(
        _tc_mid_body,
        out_shape=jax.ShapeDtypeStruct((N // 2, 2 * EMB), jnp.float32),
    )(tp, p_prev, dinv, b, w)


def _tc_fin_body(t_ref, pp_ref, dinvp_ref, b_ref, h_ref):
    t = t_ref[0] + t_ref[1] + pp_ref[...]
    dinvp = dinvp_ref[...]
    bp = jnp.concatenate([b_ref[...], b_ref[...]])
    h_ref[...] = jnp.tanh(t * dinvp + bp[None, :])


def _tc_fin(tp, p_prev, dinv, b):
    return pl.pallas_call(
        _tc_fin_body,
        out_shape=jax.ShapeDtypeStruct((N // 2, 2 * EMB), jnp.float32),
    )(tp, p_prev, dinv, b)


def _tc_post_body(maxp_ref, sump_ref, cntp_ref, wout_ref, bout_ref, out_ref, hid_ref):
    gmax = jnp.max(maxp_ref[...], axis=0)
    gsum = jnp.sum(sump_ref[...], axis=0)
    cnt = jnp.sum(cntp_ref[...], axis=0)
    gmean = gsum / jnp.clip(cnt, 1.0)[:, None]
    hidden = jnp.concatenate([gmax, gmean], axis=1)
    hid_ref[...] = hidden
    out_ref[...] = (
        jnp.dot(hidden, wout_ref[...], preferred_element_type=jnp.float32)
        + bout_ref[...][None, :]
    )


def _tc_post(maxp, sump, cntp, wout, bout):
    return pl.pallas_call(
        _tc_post_body,
        out_shape=[
            jax.ShapeDtypeStruct((G, 1), jnp.float32),
            jax.ShapeDtypeStruct((G, 2 * EMB), jnp.float32),
        ],
    )(maxp, sump, cntp, wout, bout)


def _tc_edges_body(e_ref, r_ref, c_ref):
    rows = e_ref[0, :].reshape(E // BLK, BLK)
    cols = e_ref[1, :].reshape(E // BLK, BLK)
    npad = E_PAD - E
    padr = (jnp.arange(npad, dtype=jnp.int32) % N).reshape(npad // BLK, BLK)
    padc = N + (jnp.arange(npad, dtype=jnp.int32) % (NPAD - N)).reshape(npad // BLK, BLK)
    r_ref[...] = jnp.concatenate([rows, padr], axis=0)
    c_ref[...] = jnp.concatenate([cols, padc], axis=0)


def _tc_edges(edge_index):
    return pl.pallas_call(
        _tc_edges_body,
        out_shape=[
            jax.ShapeDtypeStruct((E_PAD // BLK, BLK), jnp.int32),
            jax.ShapeDtypeStruct((E_PAD // BLK, BLK), jnp.int32),
        ],
    )(edge_index)


@jax.jit
def kernel(x, edge_index, batch_index, W0, b0, W1, b1, W2, b2, Wout, bout):
    rows, cols = _tc_edges(edge_index)
    half = jnp.full((DEG_PAD,), 0.5, jnp.float32)
    degp2 = _sc_deg(cols, half).reshape(NC, DEG_PAD // 2, 2)
    xp = x.reshape(N // 2, 2 * DF)
    P0p, dinvp = _tc_pre(degp2, xp, W0)
    T0 = _sc_prop(rows, cols, P0p.reshape(N, EMB))
    P1p = _tc_mid(T0.reshape(NC, N // 2, 2 * EMB), P0p, dinvp, b0, W1)
    T1 = _sc_prop(rows, cols, P1p.reshape(N, EMB))
    P2p = _tc_mid(T1.reshape(NC, N // 2, 2 * EMB), P1p, dinvp, b1, W2)
    T2 = _sc_prop(rows, cols, P2p.reshape(N, EMB))
    h3p = _tc_fin(T2.reshape(NC, N // 2, 2 * EMB), P2p, dinvp, b2)
    maxp, sump, cntp = _sc_pool(h3p.reshape(N, EMB), batch_index)
    out, hidden = _tc_post(maxp, sump, cntp, Wout, bout)
    return out, hidden
